# Initial kernel scaffold; baseline (speedup 1.0000x reference)
import jax
import jax.numpy as jnp
from jax.experimental import pallas as pl


def kernel(queries, keys, values, attn_mask):
    raise NotImplementedError("write your pallas kernel here")


# speedup vs baseline: 1.6488x; 1.6488x over previous
import functools
import math

import jax
import jax.numpy as jnp
from jax.experimental import pallas as pl
from jax.experimental.pallas import tpu as pltpu


def _corr_body(TM, TN, q_ref, k_ref, out_ref, acc_ref):
    s = pl.program_id(1)
    j = pl.program_id(2)
    nj = pl.num_programs(2)
    prod = jax.lax.dot_general(
        q_ref[0], k_ref[0], (((1,), (1,)), ((), ())),
        preferred_element_type=jnp.float32,
        precision=jax.lax.Precision.HIGHEST)
    acc_ref[:, pl.ds(j * TN, TN)] = prod

    @pl.when(j == nj - 1)
    def _reduce():
        x = acc_ref[:, :]
        while x.shape[0] > 1:
            h = x.shape[0] // 2
            x = x[:h] + jnp.roll(x[h:], h, axis=-1)
        contrib = jnp.roll(x, 1, axis=-1)

        @pl.when(s == 0)
        def _init():
            out_ref[0] = contrib

        @pl.when(s > 0)
        def _acc():
            out_ref[0] = out_ref[0] + contrib


def _topk_body(K, L, HE, B, mv_ref, idx_ref, w_ref):
    mv = mv_ref[...]
    sel = jnp.sum(mv, axis=0, keepdims=True)
    iota = jax.lax.broadcasted_iota(jnp.int32, (1, L), 1)
    iota_k = jax.lax.broadcasted_iota(jnp.int32, (1, K), 1)
    idx_vec = jnp.zeros((1, K), jnp.int32)
    w_mat = jnp.zeros((B, K), jnp.float32)
    for i in range(K):
        m = jnp.max(sel)
        ii = jnp.min(jnp.where(sel == m, iota, L))
        idx_vec = jnp.where(iota_k == i, ii, idx_vec)
        wcol = jnp.sum(jnp.where(iota == ii, mv, 0.0), axis=1, keepdims=True)
        w_mat = jnp.where(iota_k == i, wcol, w_mat)
        sel = jnp.where(iota == ii, -jnp.inf, sel)
    idx_ref[...] = idx_vec
    w = w_mat * (1.0 / HE)
    w = jnp.exp(w - jnp.max(w, axis=1, keepdims=True))
    w_ref[...] = w / jnp.sum(w, axis=1, keepdims=True)


def _agg_body(TB, L, K, idx_ref, w_ref, vext_ref, out_ref, vbuf, sems):
    b = pl.program_id(0)
    lb = pl.program_id(1)
    l0 = lb * TB
    copies = []
    for i in range(K):
        start = l0 + idx_ref[i]
        start = jnp.where(start >= L, start - L, start)
        cp = pltpu.make_async_copy(
            vext_ref.at[b, pl.ds(start, TB), :, :], vbuf.at[i], sems.at[i])
        cp.start()
        copies.append(cp)
    acc = None
    for i in range(K):
        copies[i].wait()
        term = vbuf[i] * w_ref[b, i]
        acc = term if acc is None else acc + term
    out_ref[0] = acc


def kernel(queries, keys, values, attn_mask):
    B, L, H, E = queries.shape
    D = H * E
    K = int(1 * math.log(L))
    TM = TN = 256 if L % 256 == 0 else 64
    TB = 256 if L % 256 == 0 else 64
    nblk = L // TN

    qf = queries.reshape(B, L, D)
    kf = jnp.flip(keys.reshape(B, L, D), axis=1)
    vf = values.reshape(B, L, D)
    lane = D // 8
    vext = jnp.concatenate([vf, vf[:, :TB]], axis=1).reshape(B, L + TB, 8, lane)

    mv = pl.pallas_call(
        functools.partial(_corr_body, TM, TN),
        grid=(B, L // TM, nblk),
        in_specs=[
            pl.BlockSpec((1, TM, D), lambda b, s, j: (b, s, 0)),
            pl.BlockSpec((1, TN, D), lambda b, s, j: (b, (j - s) % nblk, 0)),
        ],
        out_specs=pl.BlockSpec((1, 1, L), lambda b, s, j: (b, 0, 0)),
        out_shape=jax.ShapeDtypeStruct((B, 1, L), jnp.float32),
        scratch_shapes=[pltpu.VMEM((TM, L), jnp.float32)],
        compiler_params=pltpu.CompilerParams(
            dimension_semantics=("arbitrary", "arbitrary", "arbitrary")),
    )(qf, kf)
    mv = mv.reshape(B, L)

    idx, w = pl.pallas_call(
        functools.partial(_topk_body, K, L, D, B),
        in_specs=[pl.BlockSpec((B, L), lambda: (0, 0))],
        out_specs=[
            pl.BlockSpec((1, K), lambda: (0, 0)),
            pl.BlockSpec((B, K), lambda: (0, 0)),
        ],
        out_shape=[
            jax.ShapeDtypeStruct((1, K), jnp.int32),
            jax.ShapeDtypeStruct((B, K), jnp.float32),
        ],
    )(mv)

    out = pl.pallas_call(
        functools.partial(_agg_body, TB, L, K),
        grid_spec=pltpu.PrefetchScalarGridSpec(
            num_scalar_prefetch=2,
            grid=(B, L // TB),
            in_specs=[pl.BlockSpec(memory_space=pl.MemorySpace.ANY)],
            out_specs=pl.BlockSpec(
                (1, TB, 8, lane), lambda b, lb, i_ref, w_ref: (b, lb, 0, 0)),
            scratch_shapes=[
                pltpu.VMEM((K, TB, 8, lane), jnp.float32),
                pltpu.SemaphoreType.DMA((K,)),
            ],
        ),
        out_shape=jax.ShapeDtypeStruct((B, L, 8, lane), jnp.float32),
    )(idx.reshape(K), w, vext)

    return out.reshape(B, L, H, E)


# speedup vs baseline: 1.8540x; 1.1244x over previous
import functools
import math

import jax
import jax.numpy as jnp
from jax.experimental import pallas as pl
from jax.experimental.pallas import tpu as pltpu


def _corr_body(TM, TN, qhi_ref, qlo_ref, khi_ref, klo_ref, out_ref, acc_ref):
    s = pl.program_id(1)
    j = pl.program_id(2)
    nj = pl.num_programs(2)

    def mm(a_ref, b_ref):
        return jax.lax.dot_general(
            a_ref[0], b_ref[0], (((1,), (1,)), ((), ())),
            preferred_element_type=jnp.float32)

    prod = mm(qhi_ref, khi_ref) + mm(qhi_ref, klo_ref) + mm(qlo_ref, khi_ref)
    acc_ref[:, pl.ds(j * TN, TN)] = prod

    @pl.when(j == nj - 1)
    def _reduce():
        x = acc_ref[:, :]
        while x.shape[0] > 1:
            h = x.shape[0] // 2
            x = x[:h] + jnp.roll(x[h:], h, axis=-1)
        contrib = jnp.roll(x, 1, axis=-1)

        @pl.when(s == 0)
        def _init():
            out_ref[0] = contrib

        @pl.when(s > 0)
        def _acc():
            out_ref[0] = out_ref[0] + contrib


def _topk_body(K, L, HE, B, mv_ref, idx_ref, w_ref):
    mv = mv_ref[...]
    sel = jnp.sum(mv, axis=0, keepdims=True)
    iota = jax.lax.broadcasted_iota(jnp.int32, (1, L), 1)
    iota_k = jax.lax.broadcasted_iota(jnp.int32, (1, K), 1)
    idx_vec = jnp.zeros((1, K), jnp.int32)
    w_mat = jnp.zeros((B, K), jnp.float32)
    for i in range(K):
        m = jnp.max(sel)
        ii = jnp.min(jnp.where(sel == m, iota, L))
        idx_vec = jnp.where(iota_k == i, ii, idx_vec)
        wcol = jnp.sum(jnp.where(iota == ii, mv, 0.0), axis=1, keepdims=True)
        w_mat = jnp.where(iota_k == i, wcol, w_mat)
        sel = jnp.where(iota == ii, -jnp.inf, sel)
    idx_ref[...] = idx_vec
    w = w_mat * (1.0 / HE)
    w = jnp.exp(w - jnp.max(w, axis=1, keepdims=True))
    w_ref[...] = w / jnp.sum(w, axis=1, keepdims=True)


def _agg_body(TB, L, K, idx_ref, w_ref, vext_ref, out_ref, vbuf, sems):
    b = pl.program_id(0)
    lb = pl.program_id(1)
    l0 = lb * TB
    copies = []
    for i in range(K):
        start = l0 + idx_ref[i]
        start = jnp.where(start >= L, start - L, start)
        cp = pltpu.make_async_copy(
            vext_ref.at[b, pl.ds(start, TB), :, :], vbuf.at[i], sems.at[i])
        cp.start()
        copies.append(cp)
    acc = None
    for i in range(K):
        copies[i].wait()
        term = vbuf[i] * w_ref[b, i]
        acc = term if acc is None else acc + term
    out_ref[0] = acc


def kernel(queries, keys, values, attn_mask):
    B, L, H, E = queries.shape
    D = H * E
    K = int(1 * math.log(L))
    TM = TN = 256 if L % 256 == 0 else 64
    TB = 256 if L % 256 == 0 else 64
    nblk = L // TN

    qf = queries.reshape(B, L, D)
    kf = jnp.flip(keys.reshape(B, L, D), axis=1)
    qhi = qf.astype(jnp.bfloat16)
    qlo = (qf - qhi.astype(jnp.float32)).astype(jnp.bfloat16)
    khi = kf.astype(jnp.bfloat16)
    klo = (kf - khi.astype(jnp.float32)).astype(jnp.bfloat16)
    vf = values.reshape(B, L, D)
    lane = D // 8
    vext = jnp.concatenate([vf, vf[:, :TB]], axis=1).reshape(B, L + TB, 8, lane)

    mv = pl.pallas_call(
        functools.partial(_corr_body, TM, TN),
        grid=(B, L // TM, nblk),
        in_specs=[
            pl.BlockSpec((1, TM, D), lambda b, s, j: (b, s, 0)),
            pl.BlockSpec((1, TM, D), lambda b, s, j: (b, s, 0)),
            pl.BlockSpec((1, TN, D), lambda b, s, j: (b, (j - s) % nblk, 0)),
            pl.BlockSpec((1, TN, D), lambda b, s, j: (b, (j - s) % nblk, 0)),
        ],
        out_specs=pl.BlockSpec((1, 1, L), lambda b, s, j: (b, 0, 0)),
        out_shape=jax.ShapeDtypeStruct((B, 1, L), jnp.float32),
        scratch_shapes=[pltpu.VMEM((TM, L), jnp.float32)],
        compiler_params=pltpu.CompilerParams(
            dimension_semantics=("parallel", "arbitrary", "arbitrary")),
    )(qhi, qlo, khi, klo)
    mv = mv.reshape(B, L)

    idx, w = pl.pallas_call(
        functools.partial(_topk_body, K, L, D, B),
        in_specs=[pl.BlockSpec((B, L), lambda: (0, 0))],
        out_specs=[
            pl.BlockSpec((1, K), lambda: (0, 0)),
            pl.BlockSpec((B, K), lambda: (0, 0)),
        ],
        out_shape=[
            jax.ShapeDtypeStruct((1, K), jnp.int32),
            jax.ShapeDtypeStruct((B, K), jnp.float32),
        ],
    )(mv)

    out = pl.pallas_call(
        functools.partial(_agg_body, TB, L, K),
        grid_spec=pltpu.PrefetchScalarGridSpec(
            num_scalar_prefetch=2,
            grid=(B, L // TB),
            in_specs=[pl.BlockSpec(memory_space=pl.MemorySpace.ANY)],
            out_specs=pl.BlockSpec(
                (1, TB, 8, lane), lambda b, lb, i_ref, w_ref: (b, lb, 0, 0)),
            scratch_shapes=[
                pltpu.VMEM((K, TB, 8, lane), jnp.float32),
                pltpu.SemaphoreType.DMA((K,)),
            ],
        ),
        out_shape=jax.ShapeDtypeStruct((B, L, 8, lane), jnp.float32),
    )(idx.reshape(K), w, vext)

    return out.reshape(B, L, H, E)


# speedup vs baseline: 2.5542x; 1.3777x over previous
import functools
import math

import jax
import jax.numpy as jnp
from jax.experimental import pallas as pl
from jax.experimental.pallas import tpu as pltpu


def _lag_tree(x, unit):
    while x.shape[0] > 1:
        h = x.shape[0] // 2
        x = x[:h] + jnp.roll(x[h:], h * unit, axis=-1)
    return x


def _corr_body(TM, qhi_ref, qlo_ref, khi_ref, klo_ref, out_ref, strip_ref):
    s = pl.program_id(1)
    ns = pl.num_programs(1)

    def mm(a_ref, b_ref):
        return jax.lax.dot_general(
            a_ref[0], b_ref[0], (((1,), (1,)), ((), ())),
            preferred_element_type=jnp.float32)

    prod = mm(qhi_ref, khi_ref) + mm(qhi_ref, klo_ref) + mm(qlo_ref, khi_ref)
    strip_ref[pl.ds(s, 1), :] = _lag_tree(prod, 1)

    @pl.when(s == ns - 1)
    def _reduce():
        out_ref[0] = jnp.roll(_lag_tree(strip_ref[:, :], TM), 1, axis=-1)


def _topk_body(K, L, HE, B, mv_ref, idx_ref, w_ref):
    mv = mv_ref[...]
    sel = jnp.sum(mv, axis=0, keepdims=True)
    iota = jax.lax.broadcasted_iota(jnp.int32, (1, L), 1)
    iota_k = jax.lax.broadcasted_iota(jnp.int32, (1, K), 1)
    idx_vec = jnp.zeros((1, K), jnp.int32)
    w_mat = jnp.zeros((B, K), jnp.float32)
    for i in range(K):
        m = jnp.max(sel)
        ii = jnp.min(jnp.where(sel == m, iota, L))
        idx_vec = jnp.where(iota_k == i, ii, idx_vec)
        wcol = jnp.sum(jnp.where(iota == ii, mv, 0.0), axis=1, keepdims=True)
        w_mat = jnp.where(iota_k == i, wcol, w_mat)
        sel = jnp.where(iota == ii, -jnp.inf, sel)
    idx_ref[...] = idx_vec
    w = w_mat * (1.0 / HE)
    w = jnp.exp(w - jnp.max(w, axis=1, keepdims=True))
    w_ref[...] = w / jnp.sum(w, axis=1, keepdims=True)


def _agg_body(TB, L, K, idx_ref, w_ref, vext_ref, out_ref, vbuf, sems):
    b = pl.program_id(0)
    lb = pl.program_id(1)
    l0 = lb * TB
    copies = []
    for i in range(K):
        start = l0 + idx_ref[i]
        start = jnp.where(start >= L, start - L, start)
        cp = pltpu.make_async_copy(
            vext_ref.at[b, pl.ds(start, TB), :, :], vbuf.at[i], sems.at[i])
        cp.start()
        copies.append(cp)
    acc = None
    for i in range(K):
        copies[i].wait()
        term = vbuf[i] * w_ref[b, i]
        acc = term if acc is None else acc + term
    out_ref[0] = acc


def kernel(queries, keys, values, attn_mask):
    B, L, H, E = queries.shape
    D = H * E
    K = int(1 * math.log(L))
    TM = TN = 256 if L % 256 == 0 else 64
    TB = 256 if L % 256 == 0 else 64
    nblk = L // TN

    qf = queries.reshape(B, L, D)
    kf = jnp.flip(keys.reshape(B, L, D), axis=1)
    qhi = qf.astype(jnp.bfloat16)
    qlo = (qf - qhi.astype(jnp.float32)).astype(jnp.bfloat16)
    khi = kf.astype(jnp.bfloat16)
    klo = (kf - khi.astype(jnp.float32)).astype(jnp.bfloat16)
    vf = values.reshape(B, L, D)
    lane = D // 8
    vext = jnp.concatenate([vf, vf[:, :TB]], axis=1).reshape(B, L + TB, 8, lane)

    mv = pl.pallas_call(
        functools.partial(_corr_body, TM),
        grid=(B, L // TM),
        in_specs=[
            pl.BlockSpec((1, TM, D), lambda b, s: (b, s, 0)),
            pl.BlockSpec((1, TM, D), lambda b, s: (b, s, 0)),
            pl.BlockSpec((1, L, D), lambda b, s: (b, 0, 0)),
            pl.BlockSpec((1, L, D), lambda b, s: (b, 0, 0)),
        ],
        out_specs=pl.BlockSpec((1, 1, L), lambda b, s: (b, 0, 0)),
        out_shape=jax.ShapeDtypeStruct((B, 1, L), jnp.float32),
        scratch_shapes=[pltpu.VMEM((L // TM, L), jnp.float32)],
        compiler_params=pltpu.CompilerParams(
            dimension_semantics=("parallel", "arbitrary")),
    )(qhi, qlo, khi, klo)
    mv = mv.reshape(B, L)

    idx, w = pl.pallas_call(
        functools.partial(_topk_body, K, L, D, B),
        in_specs=[pl.BlockSpec((B, L), lambda: (0, 0))],
        out_specs=[
            pl.BlockSpec((1, K), lambda: (0, 0)),
            pl.BlockSpec((B, K), lambda: (0, 0)),
        ],
        out_shape=[
            jax.ShapeDtypeStruct((1, K), jnp.int32),
            jax.ShapeDtypeStruct((B, K), jnp.float32),
        ],
    )(mv)

    out = pl.pallas_call(
        functools.partial(_agg_body, TB, L, K),
        grid_spec=pltpu.PrefetchScalarGridSpec(
            num_scalar_prefetch=2,
            grid=(B, L // TB),
            in_specs=[pl.BlockSpec(memory_space=pl.MemorySpace.ANY)],
            out_specs=pl.BlockSpec(
                (1, TB, 8, lane), lambda b, lb, i_ref, w_ref: (b, lb, 0, 0)),
            scratch_shapes=[
                pltpu.VMEM((K, TB, 8, lane), jnp.float32),
                pltpu.SemaphoreType.DMA((K,)),
            ],
        ),
        out_shape=jax.ShapeDtypeStruct((B, L, 8, lane), jnp.float32),
    )(idx.reshape(K), w, vext)

    return out.reshape(B, L, H, E)
